# baseline (speedup 1.0000x reference)
import functools

import jax
import jax.numpy as jnp
from jax import lax
from jax.experimental import pallas as pl
from jax.experimental.pallas import tpu as pltpu
from jax.experimental.pallas import tpu_sc as plsc

_B = 16384
_D = 64
_K = 5
_NC = 2
_NS = 16
_L = 16
_NW = _NC * _NS
_EPW = _B // _NW
_ROUNDS = _EPW // _L


def _sc_body(pos_u_hbm, pos_v_hbm, neg_hbm, uw_hbm, vw_hbm,
             pos_out, neg_out,
             s_iu, s_iv, s_in, ub, vnb,
             o_pos, o_neg, sem):
    wid = lax.axis_index("s") * _NC + lax.axis_index("c")
    base = pl.multiple_of(wid * _EPW, _EPW)
    iota = lax.iota(jnp.int32, _L)

    pltpu.sync_copy(pos_u_hbm.at[pl.ds(base, _EPW)], s_iu)
    pltpu.sync_copy(pos_v_hbm.at[pl.ds(base, _EPW)], s_iv)
    pltpu.sync_copy(neg_hbm.at[pl.ds(base * _K, _EPW * _K)], s_in)

    def _scalar(vec, i):
        return jnp.sum(jnp.where(iota == i, vec, 0))

    def round_body(r, carry):
        iu = s_iu[pl.ds(r * _L, _L)]
        iv = s_iv[pl.ds(r * _L, _L)]
        cps = []
        for i in range(_L):
            cps.append(pltpu.async_copy(
                uw_hbm.at[pl.ds(_scalar(iu, i), 1)], ub.at[pl.ds(i, 1)], sem))
            cps.append(pltpu.async_copy(
                vw_hbm.at[pl.ds(_scalar(iv, i), 1)], vnb.at[pl.ds(i, 1)], sem))
        for j in range(_K):
            cj = s_in[pl.ds(r * (_L * _K) + j * _L, _L)]
            for i in range(_L):
                cps.append(pltpu.async_copy(
                    vw_hbm.at[pl.ds(_scalar(cj, i), 1)],
                    vnb.at[pl.ds(_L + j * _L + i, 1)], sem))
        for cp in cps:
            cp.wait()

        nrows = [_L + iota * _K + k for k in range(_K)]

        def d_body(dd, accs):
            colv = jnp.zeros((_L,), jnp.int32) + dd
            u_d = plsc.load_gather(ub, [iota, colv])
            v_d = plsc.load_gather(vnb, [iota, colv])
            new = [accs[0] + u_d * v_d]
            for k in range(_K):
                n_d = plsc.load_gather(vnb, [nrows[k], colv])
                new.append(accs[k + 1] + u_d * n_d)
            return tuple(new)

        zero = jnp.zeros((_L,), jnp.float32)
        accs = lax.fori_loop(0, _D, d_body, (zero,) * (_K + 1))
        o_pos[pl.ds(r * _L, _L)] = accs[0]
        for k in range(_K):
            o_neg[pl.ds(k * _EPW + r * _L, _L)] = accs[k + 1]
        return carry

    lax.fori_loop(0, _ROUNDS, round_body, 0)
    pltpu.sync_copy(o_pos, pos_out.at[pl.ds(base, _EPW)])
    pltpu.sync_copy(o_neg, neg_out.at[pl.ds(base * _K, _EPW * _K)])


@jax.jit
def _sc_logits(pos_u, pos_v, neg_flat, uw, vw):
    mesh = plsc.VectorSubcoreMesh(core_axis_name="c", subcore_axis_name="s",
                                  num_cores=_NC, num_subcores=_NS)
    kfn = pl.kernel(
        _sc_body,
        out_type=(jax.ShapeDtypeStruct((_B,), jnp.float32),
                  jax.ShapeDtypeStruct((_B * _K,), jnp.float32)),
        mesh=mesh,
        scratch_types=[
            pltpu.VMEM((_EPW,), jnp.int32),
            pltpu.VMEM((_EPW,), jnp.int32),
            pltpu.VMEM((_EPW * _K,), jnp.int32),
            pltpu.VMEM((_L, _D), jnp.float32),
            pltpu.VMEM(((_K + 1) * _L, _D), jnp.float32),
            pltpu.VMEM((_EPW,), jnp.float32),
            pltpu.VMEM((_EPW * _K,), jnp.float32),
            pltpu.SemaphoreType.DMA,
        ],
        compiler_params=pltpu.CompilerParams(needs_layout_passes=False,
                                             use_tc_tiling_on_sc=False),
    )
    return kfn(pos_u, pos_v, neg_flat, uw, vw)


def _loss_body(pos_ref, neg_ref, out_ref):
    total = (jnp.sum(jax.nn.log_sigmoid(pos_ref[...]))
             + jnp.sum(jax.nn.log_sigmoid(-neg_ref[...])))
    out_ref[...] = jnp.reshape(total, (1, 1))


def _tc_loss(pos_logits, neg_logits, interpret=False):
    return pl.pallas_call(
        _loss_body,
        out_shape=jax.ShapeDtypeStruct((1, 1), jnp.float32),
        interpret=interpret,
    )(pos_logits, neg_logits)


def kernel(pos_u, pos_v, neg_v, batch_size, u_weight, v_weight):
    pos_u = pos_u.astype(jnp.int32)
    pos_v = pos_v.astype(jnp.int32)
    neg_flat = neg_v.astype(jnp.int32).reshape(-1)
    pos_logits, neg_logits = _sc_logits(pos_u, pos_v, neg_flat,
                                        u_weight, v_weight)
    total = _tc_loss(pos_logits.reshape(_B // 128, 128),
                     neg_logits.reshape(_B * _K // 128, 128))
    return (-total[0, 0] / batch_size).astype(jnp.float32)


# speedup vs baseline: 2.0374x; 2.0374x over previous
import functools

import jax
import jax.numpy as jnp
from jax import lax
from jax.experimental import pallas as pl
from jax.experimental.pallas import tpu as pltpu
from jax.experimental.pallas import tpu_sc as plsc

_B = 16384
_D = 64
_K = 5
_NC = 2
_NS = 16
_L = 16
_NW = _NC * _NS
_EPW = _B // _NW
_ROUNDS = _EPW // _L


def _sc_body(pos_u_hbm, pos_v_hbm, neg_hbm, uw_hbm, vw_hbm,
             pos_out, neg_out,
             s_iu, s_iv, s_in, ub, vnb,
             o_pos, o_neg, sem):
    wid = lax.axis_index("s") * _NC + lax.axis_index("c")
    base = pl.multiple_of(wid * _EPW, _EPW)
    iota = lax.iota(jnp.int32, _L)

    pltpu.sync_copy(pos_u_hbm.at[pl.ds(base, _EPW)], s_iu)
    pltpu.sync_copy(pos_v_hbm.at[pl.ds(base, _EPW)], s_iv)
    pltpu.sync_copy(neg_hbm.at[pl.ds(base * _K, _EPW * _K)], s_in)

    def _scalar(vec, i):
        return jnp.sum(jnp.where(iota == i, vec, 0))

    def round_body(r, carry):
        iu = s_iu[pl.ds(r * _L, _L)]
        iv = s_iv[pl.ds(r * _L, _L)]
        def _row_copy(table, vec, i, dst, slot):
            s = _scalar(vec, i)
            blk = lax.shift_right_logical(s, 3)
            sub = lax.bitwise_and(s, 7)
            return pltpu.async_copy(
                table.at[blk, pl.ds(sub, 1)], dst.at[pl.ds(slot, 1)], sem)

        cps = []
        for i in range(_L):
            cps.append(_row_copy(uw_hbm, iu, i, ub, i))
            cps.append(_row_copy(vw_hbm, iv, i, vnb, i))
        for j in range(_K):
            cj = s_in[pl.ds(r * (_L * _K) + j * _L, _L)]
            for i in range(_L):
                cps.append(_row_copy(vw_hbm, cj, i, vnb, _L + j * _L + i))
        for cp in cps:
            cp.wait()

        nrows = [_L + iota * _K + k for k in range(_K)]

        def d_body(dd, accs):
            colv = jnp.zeros((_L,), jnp.int32) + dd
            u_d = plsc.load_gather(ub, [iota, colv])
            v_d = plsc.load_gather(vnb, [iota, colv])
            new = [accs[0] + u_d * v_d]
            for k in range(_K):
                n_d = plsc.load_gather(vnb, [nrows[k], colv])
                new.append(accs[k + 1] + u_d * n_d)
            return tuple(new)

        zero = jnp.zeros((_L,), jnp.float32)
        accs = lax.fori_loop(0, _D, d_body, (zero,) * (_K + 1))
        o_pos[pl.ds(r * _L, _L)] = accs[0]
        for k in range(_K):
            o_neg[pl.ds(k * _EPW + r * _L, _L)] = accs[k + 1]
        return carry

    lax.fori_loop(0, _ROUNDS, round_body, 0)
    pltpu.sync_copy(o_pos, pos_out.at[pl.ds(base, _EPW)])
    pltpu.sync_copy(o_neg, neg_out.at[pl.ds(base * _K, _EPW * _K)])


@jax.jit
def _sc_logits(pos_u, pos_v, neg_flat, uw, vw):
    mesh = plsc.VectorSubcoreMesh(core_axis_name="c", subcore_axis_name="s",
                                  num_cores=_NC, num_subcores=_NS)
    kfn = pl.kernel(
        _sc_body,
        out_type=(jax.ShapeDtypeStruct((_B,), jnp.float32),
                  jax.ShapeDtypeStruct((_B * _K,), jnp.float32)),
        mesh=mesh,
        scratch_types=[
            pltpu.VMEM((_EPW,), jnp.int32),
            pltpu.VMEM((_EPW,), jnp.int32),
            pltpu.VMEM((_EPW * _K,), jnp.int32),
            pltpu.VMEM((_L, _D), jnp.float32),
            pltpu.VMEM(((_K + 1) * _L, _D), jnp.float32),
            pltpu.VMEM((_EPW,), jnp.float32),
            pltpu.VMEM((_EPW * _K,), jnp.float32),
            pltpu.SemaphoreType.DMA,
        ],
        compiler_params=pltpu.CompilerParams(needs_layout_passes=False),
    )
    return kfn(pos_u, pos_v, neg_flat, uw, vw)


def _loss_body(pos_ref, neg_ref, out_ref):
    total = (jnp.sum(jax.nn.log_sigmoid(pos_ref[...]))
             + jnp.sum(jax.nn.log_sigmoid(-neg_ref[...])))
    out_ref[...] = jnp.reshape(total, (1, 1))


def _tc_loss(pos_logits, neg_logits, interpret=False):
    return pl.pallas_call(
        _loss_body,
        out_shape=jax.ShapeDtypeStruct((1, 1), jnp.float32),
        interpret=interpret,
    )(pos_logits, neg_logits)


def kernel(pos_u, pos_v, neg_v, batch_size, u_weight, v_weight):
    pos_u = pos_u.astype(jnp.int32)
    pos_v = pos_v.astype(jnp.int32)
    neg_flat = neg_v.astype(jnp.int32).reshape(-1)
    uw3 = u_weight.reshape(-1, 8, _D)
    vw3 = v_weight.reshape(-1, 8, _D)
    pos_logits, neg_logits = _sc_logits(pos_u, pos_v, neg_flat, uw3, vw3)
    total = _tc_loss(pos_logits.reshape(_B // 128, 128),
                     neg_logits.reshape(_B * _K // 128, 128))
    return (-total[0, 0] / batch_size).astype(jnp.float32)


# speedup vs baseline: 2.1136x; 1.0374x over previous
import functools

import jax
import jax.numpy as jnp
from jax import lax
from jax.experimental import pallas as pl
from jax.experimental.pallas import tpu as pltpu
from jax.experimental.pallas import tpu_sc as plsc

_B = 16384
_D = 64
_K = 5
_NC = 2
_NS = 16
_L = 16
_NW = _NC * _NS
_EPW = _B // _NW
_ROUNDS = _EPW // _L


def _sc_body(pos_u_hbm, pos_v_hbm, neg_hbm, uw_hbm, vw_hbm,
             pos_out, neg_out,
             s_iu, s_iv, s_in, ub_a, vnb_a, ub_b, vnb_b,
             o_pos, o_neg, sem_a, sem_b):
    wid = lax.axis_index("s") * _NC + lax.axis_index("c")
    base = pl.multiple_of(wid * _EPW, _EPW)
    iota = lax.iota(jnp.int32, _L)

    pltpu.sync_copy(pos_u_hbm.at[pl.ds(base, _EPW)], s_iu)
    pltpu.sync_copy(pos_v_hbm.at[pl.ds(base, _EPW)], s_iv)
    pltpu.sync_copy(neg_hbm.at[pl.ds(base * _K, _EPW * _K)], s_in)

    def _scalar(vec, i):
        return jnp.sum(jnp.where(iota == i, vec, 0))

    def fire_round(r, ub, vnb, sem):
        iu = s_iu[pl.ds(r * _L, _L)]
        iv = s_iv[pl.ds(r * _L, _L)]

        def _row_copy(table, vec, i, dst, slot):
            s = _scalar(vec, i)
            blk = lax.shift_right_logical(s, 3)
            sub = lax.bitwise_and(s, 7)
            pltpu.async_copy(
                table.at[blk, pl.ds(sub, 1)],
                dst.at[slot // 8, pl.ds(slot % 8, 1)], sem)

        for i in range(_L):
            _row_copy(uw_hbm, iu, i, ub, i)
            _row_copy(vw_hbm, iv, i, vnb, i)
        for j in range(_K):
            cj = s_in[pl.ds(r * (_L * _K) + j * _L, _L)]
            for i in range(_L):
                _row_copy(vw_hbm, cj, i, vnb, _L + j * _L + i)

    def wait_round(ub, vnb, sem):
        pltpu.make_async_copy(uw_hbm.at[pl.ds(0, _L // 8)], ub, sem).wait()
        pltpu.make_async_copy(
            vw_hbm.at[pl.ds(0, (_K + 1) * _L // 8)], vnb, sem).wait()

    urow = [iota // 8, iota % 8]
    nrow = [[(_L + iota * _K + k) // 8, (_L + iota * _K + k) % 8]
            for k in range(_K)]

    def compute_round(r, ub, vnb):
        def d_body(dd, accs):
            colv = jnp.zeros((_L,), jnp.int32) + dd
            u_d = plsc.load_gather(ub, [urow[0], urow[1], colv])
            v_d = plsc.load_gather(vnb, [urow[0], urow[1], colv])
            new = [accs[0] + u_d * v_d]
            for k in range(_K):
                n_d = plsc.load_gather(vnb, [nrow[k][0], nrow[k][1], colv])
                new.append(accs[k + 1] + u_d * n_d)
            return tuple(new)

        zero = jnp.zeros((_L,), jnp.float32)
        accs = lax.fori_loop(0, _D, d_body, (zero,) * (_K + 1))
        o_pos[pl.ds(r * _L, _L)] = accs[0]
        for k in range(_K):
            o_neg[pl.ds(k * _EPW + r * _L, _L)] = accs[k + 1]

    fire_round(0, ub_a, vnb_a, sem_a)

    def pair_body(h, carry):
        r0 = h * 2
        fire_round(r0 + 1, ub_b, vnb_b, sem_b)
        wait_round(ub_a, vnb_a, sem_a)
        compute_round(r0, ub_a, vnb_a)
        fire_round(jnp.minimum(r0 + 2, _ROUNDS - 1), ub_a, vnb_a, sem_a)
        wait_round(ub_b, vnb_b, sem_b)
        compute_round(r0 + 1, ub_b, vnb_b)
        return carry

    lax.fori_loop(0, _ROUNDS // 2, pair_body, 0)
    wait_round(ub_a, vnb_a, sem_a)

    pltpu.sync_copy(o_pos, pos_out.at[pl.ds(base, _EPW)])
    pltpu.sync_copy(o_neg, neg_out.at[pl.ds(base * _K, _EPW * _K)])


@jax.jit
def _sc_logits(pos_u, pos_v, neg_flat, uw, vw):
    mesh = plsc.VectorSubcoreMesh(core_axis_name="c", subcore_axis_name="s",
                                  num_cores=_NC, num_subcores=_NS)
    kfn = pl.kernel(
        _sc_body,
        out_type=(jax.ShapeDtypeStruct((_B,), jnp.float32),
                  jax.ShapeDtypeStruct((_B * _K,), jnp.float32)),
        mesh=mesh,
        scratch_types=[
            pltpu.VMEM((_EPW,), jnp.int32),
            pltpu.VMEM((_EPW,), jnp.int32),
            pltpu.VMEM((_EPW * _K,), jnp.int32),
            pltpu.VMEM((_L // 8, 8, _D), jnp.float32),
            pltpu.VMEM(((_K + 1) * _L // 8, 8, _D), jnp.float32),
            pltpu.VMEM((_L // 8, 8, _D), jnp.float32),
            pltpu.VMEM(((_K + 1) * _L // 8, 8, _D), jnp.float32),
            pltpu.VMEM((_EPW,), jnp.float32),
            pltpu.VMEM((_EPW * _K,), jnp.float32),
            pltpu.SemaphoreType.DMA,
            pltpu.SemaphoreType.DMA,
        ],
        compiler_params=pltpu.CompilerParams(needs_layout_passes=False),
    )
    return kfn(pos_u, pos_v, neg_flat, uw, vw)


def _loss_body(pos_ref, neg_ref, out_ref):
    total = (jnp.sum(jax.nn.log_sigmoid(pos_ref[...]))
             + jnp.sum(jax.nn.log_sigmoid(-neg_ref[...])))
    out_ref[...] = jnp.reshape(total, (1, 1))


def _tc_loss(pos_logits, neg_logits, interpret=False):
    return pl.pallas_call(
        _loss_body,
        out_shape=jax.ShapeDtypeStruct((1, 1), jnp.float32),
        interpret=interpret,
    )(pos_logits, neg_logits)


def kernel(pos_u, pos_v, neg_v, batch_size, u_weight, v_weight):
    pos_u = pos_u.astype(jnp.int32)
    pos_v = pos_v.astype(jnp.int32)
    neg_flat = neg_v.astype(jnp.int32).reshape(-1)
    uw3 = u_weight.reshape(-1, 8, _D)
    vw3 = v_weight.reshape(-1, 8, _D)
    pos_logits, neg_logits = _sc_logits(pos_u, pos_v, neg_flat, uw3, vw3)
    total = _tc_loss(pos_logits.reshape(_B // 128, 128),
                     neg_logits.reshape(_B * _K // 128, 128))
    return (-total[0, 0] / batch_size).astype(jnp.float32)
